# baseline (speedup 1.0000x reference)
import jax
import jax.numpy as jnp
from jax.experimental import pallas as pl
from jax.experimental.pallas import tpu as pltpu


def _sage_block_kernel(adj_ref, feat_ref, wt_ref, b_ref, out_ref):
    i = pl.program_id(0)
    bm, d = out_ref.shape
    nb = jnp.dot(adj_ref[...], feat_ref[...], preferred_element_type=jnp.float32)
    self_f = feat_ref[pl.ds(i * bm, bm), :]
    out = (
        jnp.dot(self_f, wt_ref[0:d, :], preferred_element_type=jnp.float32)
        + jnp.dot(nb, wt_ref[d : 2 * d, :], preferred_element_type=jnp.float32)
        + b_ref[...]
    )
    norm = jnp.sqrt(jnp.sum(out * out, axis=1, keepdims=True))
    out_ref[...] = out / jnp.maximum(norm, 1e-12)


def kernel(features, adj, W, b):
    n, d = features.shape
    bm = 200
    wt = W.T
    b2 = b.reshape(1, d)
    return pl.pallas_call(
        _sage_block_kernel,
        grid=(n // bm,),
        in_specs=[
            pl.BlockSpec((bm, n), lambda i: (i, 0)),
            pl.BlockSpec((n, d), lambda i: (0, 0)),
            pl.BlockSpec((2 * d, d), lambda i: (0, 0)),
            pl.BlockSpec((1, d), lambda i: (0, 0)),
        ],
        out_specs=pl.BlockSpec((bm, d), lambda i: (i, 0)),
        out_shape=jax.ShapeDtypeStruct((n, d), jnp.float32),
        compiler_params=pltpu.CompilerParams(dimension_semantics=("parallel",)),
    )(adj, features, wt, b2)


# speedup vs baseline: 1.0189x; 1.0189x over previous
import jax
import jax.numpy as jnp
from jax.experimental import pallas as pl
from jax.experimental.pallas import tpu as pltpu


def _sage_block_kernel(adj_ref, feat_ref, self_ref, wt_ref, b_ref, out_ref):
    d = out_ref.shape[1]
    nb = jnp.dot(adj_ref[...], feat_ref[...], preferred_element_type=jnp.float32)
    out = (
        jnp.dot(self_ref[...], wt_ref[0:d, :], preferred_element_type=jnp.float32)
        + jnp.dot(nb, wt_ref[d : 2 * d, :], preferred_element_type=jnp.float32)
        + b_ref[...]
    )
    norm = jnp.sqrt(jnp.sum(out * out, axis=1, keepdims=True))
    out_ref[...] = out / jnp.maximum(norm, 1e-12)


def kernel(features, adj, W, b):
    n, d = features.shape
    bm = 640
    wt = W.T
    b2 = b.reshape(1, d)
    return pl.pallas_call(
        _sage_block_kernel,
        grid=(pl.cdiv(n, bm),),
        in_specs=[
            pl.BlockSpec((bm, n), lambda i: (i, 0)),
            pl.BlockSpec((n, d), lambda i: (0, 0)),
            pl.BlockSpec((bm, d), lambda i: (i, 0)),
            pl.BlockSpec((2 * d, d), lambda i: (0, 0)),
            pl.BlockSpec((1, d), lambda i: (0, 0)),
        ],
        out_specs=pl.BlockSpec((bm, d), lambda i: (i, 0)),
        out_shape=jax.ShapeDtypeStruct((n, d), jnp.float32),
        compiler_params=pltpu.CompilerParams(
            dimension_semantics=("parallel",),
            vmem_limit_bytes=100 * 1024 * 1024,
        ),
    )(adj, features, features, wt, b2)


# speedup vs baseline: 1.0531x; 1.0336x over previous
import jax
import jax.numpy as jnp
from jax.experimental import pallas as pl
from jax.experimental.pallas import tpu as pltpu


def _sage_block_kernel(adj_ref, feat_ref, wt_ref, b_ref, out_ref):
    i = pl.program_id(0)
    bm, d = out_ref.shape
    nb = jnp.dot(adj_ref[...], feat_ref[...], preferred_element_type=jnp.float32)
    self_f = feat_ref[pl.ds(i * bm, bm), :]
    out = (
        jnp.dot(self_f, wt_ref[0:d, :], preferred_element_type=jnp.float32)
        + jnp.dot(nb, wt_ref[d : 2 * d, :], preferred_element_type=jnp.float32)
        + b_ref[...]
    )
    norm = jnp.sqrt(jnp.sum(out * out, axis=1, keepdims=True))
    out_ref[...] = out / jnp.maximum(norm, 1e-12)


def kernel(features, adj, W, b):
    n, d = features.shape
    bm = 400
    wt = W.T
    b2 = b.reshape(1, d)
    return pl.pallas_call(
        _sage_block_kernel,
        grid=(n // bm,),
        in_specs=[
            pl.BlockSpec((bm, n), lambda i: (i, 0)),
            pl.BlockSpec((n, d), lambda i: (0, 0)),
            pl.BlockSpec((2 * d, d), lambda i: (0, 0)),
            pl.BlockSpec((1, d), lambda i: (0, 0)),
        ],
        out_specs=pl.BlockSpec((bm, d), lambda i: (i, 0)),
        out_shape=jax.ShapeDtypeStruct((n, d), jnp.float32),
        compiler_params=pltpu.CompilerParams(
            dimension_semantics=("parallel",),
            vmem_limit_bytes=100 * 1024 * 1024,
        ),
    )(adj, features, wt, b2)


# speedup vs baseline: 1.0592x; 1.0058x over previous
import jax
import jax.numpy as jnp
from jax.experimental import pallas as pl
from jax.experimental.pallas import tpu as pltpu

_DN = (((1,), (1,)), ((), ()))


def _sage_block_kernel(adj_ref, feat_ref, w_ref, b_ref, out_ref):
    i = pl.program_id(0)
    bm, d = out_ref.shape
    nb = jnp.dot(adj_ref[...], feat_ref[...], preferred_element_type=jnp.float32)
    self_f = feat_ref[pl.ds(i * bm, bm), :]
    out = (
        jax.lax.dot_general(
            self_f, w_ref[:, 0:d], _DN, preferred_element_type=jnp.float32
        )
        + jax.lax.dot_general(
            nb, w_ref[:, d : 2 * d], _DN, preferred_element_type=jnp.float32
        )
        + b_ref[...]
    )
    norm = jnp.sqrt(jnp.sum(out * out, axis=1, keepdims=True))
    out_ref[...] = out / jnp.maximum(norm, 1e-12)


def kernel(features, adj, W, b):
    n, d = features.shape
    bm = 400
    b2 = b.reshape(1, d)
    return pl.pallas_call(
        _sage_block_kernel,
        grid=(n // bm,),
        in_specs=[
            pl.BlockSpec((bm, n), lambda i: (i, 0)),
            pl.BlockSpec((n, d), lambda i: (0, 0)),
            pl.BlockSpec((d, 2 * d), lambda i: (0, 0)),
            pl.BlockSpec((1, d), lambda i: (0, 0)),
        ],
        out_specs=pl.BlockSpec((bm, d), lambda i: (i, 0)),
        out_shape=jax.ShapeDtypeStruct((n, d), jnp.float32),
        compiler_params=pltpu.CompilerParams(
            dimension_semantics=("parallel",),
            vmem_limit_bytes=100 * 1024 * 1024,
        ),
    )(adj, features, W, b2)


# speedup vs baseline: 1.0667x; 1.0071x over previous
import jax
import jax.numpy as jnp
from jax.experimental import pallas as pl
from jax.experimental.pallas import tpu as pltpu

_DN = (((1,), (1,)), ((), ()))


def _sage_block_kernel(adj_ref, feat_ref, w_ref, b_ref, out_ref):
    i = pl.program_id(0)
    bm, d = out_ref.shape
    nb = jnp.dot(adj_ref[...], feat_ref[...], preferred_element_type=jnp.float32)
    self_f = feat_ref[pl.ds(i * bm, bm), :]
    out = (
        jax.lax.dot_general(
            self_f, w_ref[:, 0:d], _DN, preferred_element_type=jnp.float32
        )
        + jax.lax.dot_general(
            nb, w_ref[:, d : 2 * d], _DN, preferred_element_type=jnp.float32
        )
        + b_ref[...]
    )
    norm = jnp.sqrt(jnp.sum(out * out, axis=1, keepdims=True))
    out_ref[...] = out / jnp.maximum(norm, 1e-12)


def kernel(features, adj, W, b):
    n, d = features.shape
    bm = 400
    b2 = b.reshape(1, d)
    return pl.pallas_call(
        _sage_block_kernel,
        grid=(n // bm,),
        in_specs=[
            pl.BlockSpec((bm, n), lambda i: (i, 0)),
            pl.BlockSpec((n, d), lambda i: (0, 0)),
            pl.BlockSpec((d, 2 * d), lambda i: (0, 0)),
            pl.BlockSpec((1, d), lambda i: (0, 0)),
        ],
        out_specs=pl.BlockSpec((bm, d), lambda i: (i, 0)),
        out_shape=jax.ShapeDtypeStruct((n, d), jnp.float32),
        compiler_params=pltpu.CompilerParams(
            dimension_semantics=("arbitrary",),
            vmem_limit_bytes=100 * 1024 * 1024,
        ),
    )(adj, features, W, b2)
